# baseline (speedup 1.0000x reference)
import jax
import jax.numpy as jnp
from jax import lax
from jax.experimental import pallas as pl
from jax.experimental.pallas import tpu as pltpu
from jax.experimental.pallas import tpu_sc as plsc

NC = 2
NS = 16
NW = NC * NS
CHUNK = 128
LANES = 16
KMAX = 8
WIDE = 128


def _make_sc_agg(n_pad, ngroups, k, d, count_mode, ow=WIDE):
    rpt = n_pad // NS
    gpt = ngroups // NW
    rows = k * CHUNK
    mesh = plsc.VectorSubcoreMesh(core_axis_name="c", subcore_axis_name="s")

    scratch = [
        pltpu.VMEM_SHARED((n_pad, d), jnp.float32),
        pltpu.VMEM((rows, d), jnp.float32),
        pltpu.VMEM((rows, d), jnp.float32),
        pltpu.VMEM((gpt, 1, rows), jnp.int32),
        pltpu.VMEM((gpt, 1, rows), jnp.int32),
        pltpu.SemaphoreType.DMA,
        pltpu.SemaphoreType.DMA,
    ]

    def body(*refs):
        if count_mode:
            (row_hbm, col_hbm, out_hbm,
             acc, buf0, buf1, ridx, cidx, sem0, sem1) = refs
            p_hbm = None
        else:
            (p_hbm, row_hbm, col_hbm, out_hbm,
             acc, buf0, buf1, ridx, cidx, sem0, sem1) = refs
        c = lax.axis_index("c")
        s = lax.axis_index("s")
        g = c * NS + s

        def fill_row(i, _):
            def fill_col(j, _):
                buf1[i, pl.ds(j * LANES, LANES)] = jnp.zeros(
                    (LANES,), jnp.float32)
                if count_mode:
                    buf0[i, pl.ds(j * LANES, LANES)] = jnp.ones(
                        (LANES,), jnp.float32)
                return 0
            return lax.fori_loop(0, d // LANES, fill_col, 0)
        lax.fori_loop(0, rows, fill_row, 0)

        def zacc(i, _):
            pltpu.sync_copy(buf1.at[pl.ds(0, CHUNK)],
                            acc.at[pl.ds(s * rpt + i * CHUNK, CHUNK)])
            return 0
        lax.fori_loop(0, rpt // CHUNK, zacc, 0)

        if not count_mode:
            pltpu.sync_copy(row_hbm.at[pl.ds(g * gpt, gpt)], ridx)
        pltpu.sync_copy(col_hbm.at[pl.ds(g * gpt, gpt)], cidx)
        plsc.subcore_barrier()

        if count_mode:
            def edge(j, _):
                pltpu.async_copy(buf0, acc.at[cidx.at[j, 0]], sem0, add=True)
                return 0
            lax.fori_loop(0, gpt, edge, 0)

            def drain(j, _):
                pltpu.make_async_copy(
                    buf0, acc.at[cidx.at[j, 0]], sem0).wait()
                return 0
            lax.fori_loop(0, gpt, drain, 0)
        else:
            pltpu.async_copy(p_hbm.at[ridx.at[0, 0]], buf0, sem0)

            def pair(t, _):
                j0 = 2 * t
                pltpu.make_async_copy(p_hbm.at[ridx.at[j0, 0]], buf0, sem0).wait()

                @pl.when(j0 + 1 < gpt)
                def _():
                    pltpu.async_copy(p_hbm.at[ridx.at[j0 + 1, 0]], buf1, sem1)
                pltpu.sync_copy(buf0, acc.at[cidx.at[j0, 0]], add=True)

                @pl.when(j0 + 1 < gpt)
                def _():
                    pltpu.make_async_copy(
                        p_hbm.at[ridx.at[j0 + 1, 0]], buf1, sem1).wait()

                    @pl.when(j0 + 2 < gpt)
                    def _():
                        pltpu.async_copy(
                            p_hbm.at[ridx.at[j0 + 2, 0]], buf0, sem0)
                    pltpu.sync_copy(buf1, acc.at[cidx.at[j0 + 1, 0]], add=True)
                return 0
            lax.fori_loop(0, (gpt + 1) // 2, pair, 0)
        plsc.subcore_barrier()

        rows_slc = pl.ds(s * rpt, rpt)
        if ow < d:
            pltpu.sync_copy(acc.at[rows_slc, pl.ds(0, ow)],
                            out_hbm.at[c, rows_slc])
        elif ow == d:
            pltpu.sync_copy(acc.at[rows_slc], out_hbm.at[c, rows_slc])
        else:
            pltpu.sync_copy(acc.at[rows_slc],
                            out_hbm.at[c, rows_slc, pl.ds(0, d)])

    return pl.kernel(
        body,
        out_type=jax.ShapeDtypeStruct((NC, n_pad, ow), jnp.float32),
        mesh=mesh,
        scratch_types=scratch,
        compiler_params=pltpu.CompilerParams(use_tc_tiling_on_sc=False),
    )


def _mm1_body(x_ref, w_ref, cnt_ref, p_ref, dis_ref):
    deg = cnt_ref[0, :, 0] + cnt_ref[1, :, 0] + 1.0
    dis = lax.rsqrt(deg)[:, None]
    h = jnp.dot(x_ref[...], w_ref[...], preferred_element_type=jnp.float32)
    p_ref[...] = h * dis
    dis_ref[...] = dis


def _make_fuse_body(d):
    def _fuse_body(agg_ref, p_ref, dis_ref, b_ref, g_ref, be_ref, w_ref,
                   out_ref):
        dis = dis_ref[...]
        u = ((agg_ref[0, :, :d] + agg_ref[1, :, :d] + p_ref[...]) * dis
             + b_ref[...][None, :])
        mu = jnp.mean(u, axis=-1, keepdims=True)
        uc = u - mu
        var = jnp.mean(uc * uc, axis=-1, keepdims=True)
        v = (uc * lax.rsqrt(var + 1e-5) * g_ref[...][None, :]
             + be_ref[...][None, :])
        w = jnp.where(v > 0, v, jnp.exp(v) - 1.0)
        out_ref[...] = jnp.dot(w, w_ref[...],
                               preferred_element_type=jnp.float32) * dis
    return _fuse_body


def _fin_body(agg_ref, p_ref, dis_ref, b_ref, out_ref):
    u = ((agg_ref[0, :, :2] + agg_ref[1, :, :2] + p_ref[:, :2])
         * dis_ref[...] + b_ref[...][None, :2])
    m = jnp.max(u, axis=-1, keepdims=True)
    e = jnp.exp(u - m)
    out_ref[...] = u - m - jnp.log(jnp.sum(e, axis=-1, keepdims=True))


def kernel(x, edge_index, W1, b1, g1, be1, W2, b2, g2, be2, W3, b3):
    n, d_in = x.shape
    e = edge_index.shape[1]
    n_pad = ((n + NW * LANES - 1) // (NW * LANES)) * (NW * LANES)
    d1o, d2o, d3o = W1.shape[1], W2.shape[1], W3.shape[1]
    d3p = LANES

    row = edge_index[0].astype(jnp.int32)
    col = edge_index[1].astype(jnp.int32)
    estep = NW * CHUNK * KMAX
    e_pad = ((e + estep - 1) // estep) * estep
    if e_pad != e:
        ar = jnp.arange(e_pad - e, dtype=jnp.int32)
        row = jnp.concatenate([row, ar % jnp.int32(n)])
        col = jnp.concatenate([col, jnp.int32(n) + ar % jnp.int32(n_pad - n)])
    nchunks = e_pad // CHUNK

    def grouped(a, k):
        return a.reshape(nchunks // k, 1, k * CHUNK)

    k1, k2, k3, kc = 1, 4, KMAX, KMAX
    row_k1, col_k1 = grouped(row, k1), grouped(col, k1)
    row_k2, col_k2 = grouped(row, k2), grouped(col, k2)
    row_k3, col_k3 = grouped(row, k3), grouped(col, k3)
    col_kc = grouped(col, kc)

    w3_p = jnp.pad(W3, ((0, 0), (0, d3p - d3o)))

    cnt = _make_sc_agg(n_pad, nchunks // kc, kc, LANES, True, ow=8)(col_kc, col_kc)

    blk = 2000
    f32 = jnp.float32

    def rspec(width):
        return pl.BlockSpec((blk, width), lambda i: (i, 0))

    def aspec(w=WIDE):
        return pl.BlockSpec((NC, blk, w), lambda i: (0, i, 0))

    def vspec():
        return pl.BlockSpec((blk, 1), lambda i: (i, 0))

    def cspec(*shape):
        nz = (0,) * len(shape)
        return pl.BlockSpec(shape, lambda i, _nz=nz: _nz)

    grid = (n // blk,)

    p1, dis = pl.pallas_call(
        _mm1_body,
        grid=grid,
        in_specs=[rspec(d_in), cspec(d_in, d1o), aspec(8)],
        out_specs=[rspec(d1o), vspec()],
        out_shape=[jax.ShapeDtypeStruct((n, d1o), f32),
                   jax.ShapeDtypeStruct((n, 1), f32)],
    )(x, W1, cnt)

    agg1 = _make_sc_agg(n_pad, nchunks // k1, k1, d1o, False)(
        p1, row_k1, col_k1)

    p2 = pl.pallas_call(
        _make_fuse_body(d1o),
        grid=grid,
        in_specs=[aspec(), rspec(d1o), vspec(),
                  cspec(d1o), cspec(d1o), cspec(d1o), cspec(d1o, d2o)],
        out_specs=rspec(d2o),
        out_shape=jax.ShapeDtypeStruct((n, d2o), f32),
    )(agg1, p1, dis, b1, g1, be1, W2)

    agg2 = _make_sc_agg(n_pad, nchunks // k2, k2, d2o, False)(
        p2, row_k2, col_k2)

    p3 = pl.pallas_call(
        _make_fuse_body(d2o),
        grid=grid,
        in_specs=[aspec(), rspec(d2o), vspec(),
                  cspec(d2o), cspec(d2o), cspec(d2o), cspec(d2o, d3p)],
        out_specs=rspec(d3p),
        out_shape=jax.ShapeDtypeStruct((n, d3p), f32),
    )(agg2, p2, dis, b2, g2, be2, w3_p)

    agg3 = _make_sc_agg(n_pad, nchunks // k3, k3, d3p, False, ow=8)(
        p3, row_k3, col_k3)

    y = pl.pallas_call(
        _fin_body,
        grid=grid,
        in_specs=[aspec(8), rspec(d3p), vspec(), cspec(d3p)],
        out_specs=pl.BlockSpec((blk, d3o), lambda i: (i, 0)),
        out_shape=jax.ShapeDtypeStruct((n, d3o), f32),
    )(agg3, p3, dis, jnp.pad(b3, (0, d3p - d3o)))

    return y


# speedup vs baseline: 1.0895x; 1.0895x over previous
import jax
import jax.numpy as jnp
from jax import lax
from jax.experimental import pallas as pl
from jax.experimental.pallas import tpu as pltpu
from jax.experimental.pallas import tpu_sc as plsc

NC = 2
NS = 16
NW = NC * NS
CHUNK = 128
LANES = 16
KMAX = 8
WIDE = 128


def _make_sc_agg(n_pad, ngroups, k, d, count_mode, ow=WIDE):
    rpt = n_pad // NS
    gpt = ngroups // NW
    rows = k * CHUNK
    mesh = plsc.VectorSubcoreMesh(core_axis_name="c", subcore_axis_name="s")

    scratch = [
        pltpu.VMEM_SHARED((n_pad, d), jnp.float32),
        pltpu.VMEM((rows, d), jnp.float32),
        pltpu.VMEM((rows, d), jnp.float32),
        pltpu.VMEM((gpt, 1, rows), jnp.int32),
        pltpu.VMEM((gpt, 1, rows), jnp.int32),
        pltpu.SemaphoreType.DMA,
        pltpu.SemaphoreType.DMA,
    ]

    def body(*refs):
        if count_mode:
            (row_hbm, col_hbm, out_hbm,
             acc, buf0, buf1, ridx, cidx, sem0, sem1) = refs
            p_hbm = None
        else:
            (p_hbm, row_hbm, col_hbm, out_hbm,
             acc, buf0, buf1, ridx, cidx, sem0, sem1) = refs
        c = lax.axis_index("c")
        s = lax.axis_index("s")
        g = c * NS + s

        def fill_row(i, _):
            def fill_col(j, _):
                buf1[i, pl.ds(j * LANES, LANES)] = jnp.zeros(
                    (LANES,), jnp.float32)
                if count_mode:
                    buf0[i, pl.ds(j * LANES, LANES)] = jnp.ones(
                        (LANES,), jnp.float32)
                return 0
            return lax.fori_loop(0, d // LANES, fill_col, 0)
        lax.fori_loop(0, rows, fill_row, 0)

        def zacc(i, _):
            pltpu.sync_copy(buf1.at[pl.ds(0, CHUNK)],
                            acc.at[pl.ds(s * rpt + i * CHUNK, CHUNK)])
            return 0
        lax.fori_loop(0, rpt // CHUNK, zacc, 0)

        if not count_mode:
            pltpu.sync_copy(row_hbm.at[pl.ds(g * gpt, gpt)], ridx)
        pltpu.sync_copy(col_hbm.at[pl.ds(g * gpt, gpt)], cidx)
        plsc.subcore_barrier()

        if count_mode:
            def edge(j, _):
                pltpu.async_copy(buf0, acc.at[cidx.at[j, 0]], sem0, add=True)
                return 0
            lax.fori_loop(0, gpt, edge, 0)

            def drain(j, _):
                pltpu.make_async_copy(
                    buf0, acc.at[cidx.at[j, 0]], sem0).wait()
                return 0
            lax.fori_loop(0, gpt, drain, 0)
        else:
            pltpu.async_copy(p_hbm.at[ridx.at[0, 0]], buf0, sem0)

            def pair(t, _):
                j0 = 2 * t
                pltpu.make_async_copy(p_hbm.at[ridx.at[j0, 0]], buf0, sem0).wait()

                @pl.when(j0 + 1 < gpt)
                def _():
                    pltpu.async_copy(p_hbm.at[ridx.at[j0 + 1, 0]], buf1, sem1)
                pltpu.sync_copy(buf0, acc.at[cidx.at[j0, 0]], add=True)

                @pl.when(j0 + 1 < gpt)
                def _():
                    pltpu.make_async_copy(
                        p_hbm.at[ridx.at[j0 + 1, 0]], buf1, sem1).wait()

                    @pl.when(j0 + 2 < gpt)
                    def _():
                        pltpu.async_copy(
                            p_hbm.at[ridx.at[j0 + 2, 0]], buf0, sem0)
                    pltpu.sync_copy(buf1, acc.at[cidx.at[j0 + 1, 0]], add=True)
                return 0
            lax.fori_loop(0, (gpt + 1) // 2, pair, 0)
        plsc.subcore_barrier()

        rows_slc = pl.ds(s * rpt, rpt)
        if ow < d:
            pltpu.sync_copy(acc.at[rows_slc, pl.ds(0, ow)],
                            out_hbm.at[c, rows_slc])
        elif ow == d:
            pltpu.sync_copy(acc.at[rows_slc], out_hbm.at[c, rows_slc])
        else:
            pltpu.sync_copy(acc.at[rows_slc],
                            out_hbm.at[c, rows_slc, pl.ds(0, d)])

    return pl.kernel(
        body,
        out_type=jax.ShapeDtypeStruct((NC, n_pad, ow), jnp.float32),
        mesh=mesh,
        scratch_types=scratch,
        compiler_params=pltpu.CompilerParams(use_tc_tiling_on_sc=False),
    )


def _mm1_body(x_ref, w_ref, cnt_ref, p_ref, dis_ref):
    deg = cnt_ref[0, :, 0] + cnt_ref[1, :, 0] + 1.0
    dis = lax.rsqrt(deg)[:, None]
    h = jnp.dot(x_ref[...], w_ref[...], preferred_element_type=jnp.float32)
    p_ref[...] = h * dis
    dis_ref[...] = dis


def _make_fuse_body(d):
    def _fuse_body(agg_ref, p_ref, dis_ref, b_ref, g_ref, be_ref, w_ref,
                   out_ref):
        dis = dis_ref[...]
        u = ((agg_ref[0, :, :d] + agg_ref[1, :, :d] + p_ref[...]) * dis
             + b_ref[...][None, :])
        mu = jnp.mean(u, axis=-1, keepdims=True)
        uc = u - mu
        var = jnp.mean(uc * uc, axis=-1, keepdims=True)
        v = (uc * lax.rsqrt(var + 1e-5) * g_ref[...][None, :]
             + be_ref[...][None, :])
        w = jnp.where(v > 0, v, jnp.exp(v) - 1.0)
        out_ref[...] = jnp.dot(w, w_ref[...],
                               preferred_element_type=jnp.float32) * dis
    return _fuse_body


def _fin_body(agg_ref, p_ref, dis_ref, b_ref, out_ref):
    u = ((agg_ref[0, :, :2] + agg_ref[1, :, :2] + p_ref[:, :2])
         * dis_ref[...] + b_ref[...][None, :2])
    m = jnp.max(u, axis=-1, keepdims=True)
    e = jnp.exp(u - m)
    out_ref[...] = u - m - jnp.log(jnp.sum(e, axis=-1, keepdims=True))


def kernel(x, edge_index, W1, b1, g1, be1, W2, b2, g2, be2, W3, b3):
    n, d_in = x.shape
    e = edge_index.shape[1]
    n_pad = ((n + NW * LANES - 1) // (NW * LANES)) * (NW * LANES)
    d1o, d2o, d3o = W1.shape[1], W2.shape[1], W3.shape[1]
    d3p = LANES

    row = edge_index[0].astype(jnp.int32)
    col = edge_index[1].astype(jnp.int32)
    estep = NW * CHUNK * KMAX
    e_pad = ((e + estep - 1) // estep) * estep
    if e_pad != e:
        ar = jnp.arange(e_pad - e, dtype=jnp.int32)
        row = jnp.concatenate([row, ar % jnp.int32(n)])
        col = jnp.concatenate([col, jnp.int32(n) + ar % jnp.int32(n_pad - n)])
    nchunks = e_pad // CHUNK

    def grouped(a, k):
        return a.reshape(nchunks // k, 1, k * CHUNK)

    k1, k2, k3, kc = 1, 4, KMAX, KMAX
    row_k1, col_k1 = grouped(row, k1), grouped(col, k1)
    row_k2, col_k2 = grouped(row, k2), grouped(col, k2)
    row_k3, col_k3 = grouped(row, k3), grouped(col, k3)
    col_kc = grouped(col, kc)

    w3_p = jnp.pad(W3, ((0, 0), (0, d3p - d3o)))

    cnt = _make_sc_agg(n_pad, nchunks // kc, kc, LANES, True)(col_kc, col_kc)

    blk = 2000
    f32 = jnp.float32

    def rspec(width):
        return pl.BlockSpec((blk, width), lambda i: (i, 0))

    def aspec(w=WIDE):
        return pl.BlockSpec((NC, blk, w), lambda i: (0, i, 0))

    def vspec():
        return pl.BlockSpec((blk, 1), lambda i: (i, 0))

    def cspec(*shape):
        nz = (0,) * len(shape)
        return pl.BlockSpec(shape, lambda i, _nz=nz: _nz)

    grid = (n // blk,)

    p1, dis = pl.pallas_call(
        _mm1_body,
        grid=grid,
        in_specs=[rspec(d_in), cspec(d_in, d1o), aspec()],
        out_specs=[rspec(d1o), vspec()],
        out_shape=[jax.ShapeDtypeStruct((n, d1o), f32),
                   jax.ShapeDtypeStruct((n, 1), f32)],
    )(x, W1, cnt)

    agg1 = _make_sc_agg(n_pad, nchunks // k1, k1, d1o, False)(
        p1, row_k1, col_k1)

    p2 = pl.pallas_call(
        _make_fuse_body(d1o),
        grid=grid,
        in_specs=[aspec(), rspec(d1o), vspec(),
                  cspec(d1o), cspec(d1o), cspec(d1o), cspec(d1o, d2o)],
        out_specs=rspec(d2o),
        out_shape=jax.ShapeDtypeStruct((n, d2o), f32),
    )(agg1, p1, dis, b1, g1, be1, W2)

    agg2 = _make_sc_agg(n_pad, nchunks // k2, k2, d2o, False)(
        p2, row_k2, col_k2)

    p3 = pl.pallas_call(
        _make_fuse_body(d2o),
        grid=grid,
        in_specs=[aspec(), rspec(d2o), vspec(),
                  cspec(d2o), cspec(d2o), cspec(d2o), cspec(d2o, d3p)],
        out_specs=rspec(d3p),
        out_shape=jax.ShapeDtypeStruct((n, d3p), f32),
    )(agg2, p2, dis, b2, g2, be2, w3_p)

    agg3 = _make_sc_agg(n_pad, nchunks // k3, k3, d3p, False)(
        p3, row_k3, col_k3)

    y = pl.pallas_call(
        _fin_body,
        grid=grid,
        in_specs=[aspec(), rspec(d3p), vspec(), cspec(d3p)],
        out_specs=pl.BlockSpec((blk, d3o), lambda i: (i, 0)),
        out_shape=jax.ShapeDtypeStruct((n, d3o), f32),
    )(agg3, p3, dis, jnp.pad(b3, (0, d3p - d3o)))

    return y
